# baseline (speedup 1.0000x reference)
import functools

import jax
import jax.numpy as jnp
from jax.experimental import pallas as pl
from jax.experimental.pallas import tpu as pltpu

_B, _T, _D = 16, 1024, 32
_K = 8192
_TT = 128
_BT = _B * _T
_G = _BT // _TT

_PREC = jax.lax.Precision.DEFAULT


def _dot(a, b, dims):
    return jax.lax.dot_general(
        a, b, (dims, ((), ())),
        preferred_element_type=jnp.float32, precision=_PREC)


def _body(x_ref, cb0_ref, c2_ref, cbint_ref, pint_ref, recon_ref, loss_ref):
    x = x_ref[...]
    cb0 = cb0_ref[0]
    c2 = c2_ref[...]
    cb_int = cbint_ref[...]

    n0 = jnp.sum(cb0 * cb0, axis=1)[None, :]
    s0 = 2.0 * _dot(x, cb0, ((1,), (1,))) - n0
    m0 = jnp.max(s0, axis=1, keepdims=True)
    e0 = jnp.exp(s0 - m0)
    d0 = jnp.sum(e0, axis=1, keepdims=True)
    q0 = _dot(e0, cb0, ((1,), (0,))) * (1.0 / d0)
    r1 = x - q0

    a = jnp.concatenate([x, r1], axis=1)
    n_int = 0.25 * jnp.sum(c2 * c2, axis=0)[None, :]
    s = _dot(a, c2, ((1,), (0,))) - n_int
    lane = jax.lax.broadcasted_iota(jnp.int32, (1, 2 * _K), 1)
    even = (lane % 2) == 0
    neg = jnp.float32(-jnp.inf)
    m_e = jnp.max(jnp.where(even, s, neg), axis=1, keepdims=True)
    m_o = jnp.max(jnp.where(even, neg, s), axis=1, keepdims=True)
    e = jnp.exp(s - jnp.where(even, m_e, m_o))
    d_e = jnp.sum(jnp.where(even, e, 0.0), axis=1, keepdims=True)
    d_o = jnp.sum(jnp.where(even, 0.0, e), axis=1, keepdims=True)
    p = e * jnp.where(even, 1.0 / d_e, 1.0 / d_o)
    pint_ref[...] = p

    recon = _dot(p, cb_int, ((1,), (0,)))
    recon_ref[...] = recon
    dr = recon - x
    loss_ref[...] = (jnp.sum(r1 * r1) + jnp.sum(dr * dr)).reshape(1, 1, 1)


@jax.jit
def kernel(x, codebooks):
    x2 = x.reshape(_BT, _D)
    cb = codebooks
    cb_int = jnp.swapaxes(cb, 0, 1).reshape(2 * _K, _D)
    z = jnp.zeros((_K, _D), dtype=cb.dtype)
    top = jnp.stack([cb[0], z], axis=1).reshape(2 * _K, _D)
    bot = jnp.stack([z, cb[1]], axis=1).reshape(2 * _K, _D)
    c2 = 2.0 * jnp.concatenate([top, bot], axis=1).T

    pint, recon2, losses = pl.pallas_call(
        _body,
        grid=(_G,),
        in_specs=[
            pl.BlockSpec((_TT, _D), lambda i: (i, 0)),
            pl.BlockSpec((1, _K, _D), lambda i: (0, 0, 0)),
            pl.BlockSpec((2 * _D, 2 * _K), lambda i: (0, 0)),
            pl.BlockSpec((2 * _K, _D), lambda i: (0, 0)),
        ],
        out_specs=[
            pl.BlockSpec((_TT, 2 * _K), lambda i: (i, 0)),
            pl.BlockSpec((_TT, _D), lambda i: (i, 0)),
            pl.BlockSpec((1, 1, 1), lambda i: (i, 0, 0)),
        ],
        out_shape=[
            jax.ShapeDtypeStruct((_BT, 2 * _K), jnp.float32),
            jax.ShapeDtypeStruct((_BT, _D), jnp.float32),
            jax.ShapeDtypeStruct((_G, 1, 1), jnp.float32),
        ],
        compiler_params=pltpu.CompilerParams(
            dimension_semantics=("parallel",),
        ),
    )(x2, cb[0:1], c2, cb_int)
    index_probs = pint.reshape(_B, _T, _K, 2)
    recon = recon2.reshape(_B, _T, _D)
    loss = jnp.sum(losses) * (1.25 / (_BT * _D))
    return recon, index_probs, loss


# speedup vs baseline: 2.7490x; 2.7490x over previous
import functools

import jax
import jax.numpy as jnp
from jax.experimental import pallas as pl
from jax.experimental.pallas import tpu as pltpu

_B, _T, _D = 16, 1024, 32
_K = 8192
_TT = 128
_BT = _B * _T
_G = _BT // _TT
_NG = _K // 128

_PREC = jax.lax.Precision.DEFAULT


def _dot(a, b, dims):
    return jax.lax.dot_general(
        a, b, (dims, ((), ())),
        preferred_element_type=jnp.float32, precision=_PREC)


def _body(x_ref, cb0_ref, c2_ref, cbint_ref, pint_ref, recon_ref, loss_ref):
    x = x_ref[...]
    cb0 = cb0_ref[0]
    c2 = c2_ref[...]
    cb_int = cbint_ref[...]

    n0 = jnp.sum(cb0 * cb0, axis=1)[None, :]
    s0 = 2.0 * _dot(x, cb0, ((1,), (1,))) - n0
    m0 = jnp.max(s0, axis=1, keepdims=True)
    e0 = jnp.exp(s0 - m0)
    d0 = jnp.sum(e0, axis=1, keepdims=True)
    q0 = _dot(e0, cb0, ((1,), (0,))) * (1.0 / d0)
    r1 = x - q0

    a = jnp.concatenate([x, r1], axis=1)
    n_int = 0.25 * jnp.sum(c2 * c2, axis=0)[None, :]
    s = _dot(a, c2, ((1,), (0,))) - n_int
    lane = jax.lax.broadcasted_iota(jnp.int32, (1, 2 * _K), 1)
    even = ((lane >> 7) & 1) == 0
    neg = jnp.float32(-jnp.inf)
    m_e = jnp.max(jnp.where(even, s, neg), axis=1, keepdims=True)
    m_o = jnp.max(jnp.where(even, neg, s), axis=1, keepdims=True)
    e = jnp.exp(s - jnp.where(even, m_e, m_o))
    d_e = jnp.sum(jnp.where(even, e, 0.0), axis=1, keepdims=True)
    d_o = jnp.sum(jnp.where(even, 0.0, e), axis=1, keepdims=True)
    p = e * jnp.where(even, 1.0 / d_e, 1.0 / d_o)
    pint_ref[...] = p.reshape(_TT, 128, 128)

    recon = _dot(p, cb_int, ((1,), (0,)))
    recon_ref[...] = recon
    dr = recon - x
    loss_ref[...] = (jnp.sum(r1 * r1) + jnp.sum(dr * dr)).reshape(1, 1, 1)


@jax.jit
def kernel(x, codebooks):
    x2 = x.reshape(_BT, _D)
    cb = codebooks
    zg = jnp.zeros((_NG, 128, _D), dtype=cb.dtype)
    cb0g = cb[0].reshape(_NG, 128, _D)
    cb1g = cb[1].reshape(_NG, 128, _D)
    cb_int = jnp.stack([cb0g, cb1g], axis=1).reshape(2 * _K, _D)
    top = jnp.stack([cb0g, zg], axis=1).reshape(2 * _K, _D)
    bot = jnp.stack([zg, cb1g], axis=1).reshape(2 * _K, _D)
    c2 = 2.0 * jnp.concatenate([top, bot], axis=1).T

    pint, recon2, losses = pl.pallas_call(
        _body,
        grid=(_G,),
        in_specs=[
            pl.BlockSpec((_TT, _D), lambda i: (i, 0)),
            pl.BlockSpec((1, _K, _D), lambda i: (0, 0, 0)),
            pl.BlockSpec((2 * _D, 2 * _K), lambda i: (0, 0)),
            pl.BlockSpec((2 * _K, _D), lambda i: (0, 0)),
        ],
        out_specs=[
            pl.BlockSpec((_TT, 128, 128), lambda i: (i, 0, 0)),
            pl.BlockSpec((_TT, _D), lambda i: (i, 0)),
            pl.BlockSpec((1, 1, 1), lambda i: (i, 0, 0)),
        ],
        out_shape=[
            jax.ShapeDtypeStruct((_BT, 128, 128), jnp.float32),
            jax.ShapeDtypeStruct((_BT, _D), jnp.float32),
            jax.ShapeDtypeStruct((_G, 1, 1), jnp.float32),
        ],
        compiler_params=pltpu.CompilerParams(
            dimension_semantics=("parallel",),
        ),
    )(x2, cb[0:1], c2, cb_int)
    index_probs = (pint.reshape(_B, _T, _NG, 2, 128)
                   .transpose(0, 1, 2, 4, 3)
                   .reshape(_B, _T, _K, 2))
    recon = recon2.reshape(_B, _T, _D)
    loss = jnp.sum(losses) * (1.25 / (_BT * _D))
    return recon, index_probs, loss


# speedup vs baseline: 3.4940x; 1.2710x over previous
import functools

import jax
import jax.numpy as jnp
from jax.experimental import pallas as pl
from jax.experimental.pallas import tpu as pltpu

_B, _T, _D = 16, 1024, 32
_K = 8192
_TT = 128
_BT = _B * _T
_G = _BT // _TT
_NG = _K // 128

_PREC = jax.lax.Precision.DEFAULT


def _dot(a, b, dims):
    return jax.lax.dot_general(
        a, b, (dims, ((), ())),
        preferred_element_type=jnp.float32, precision=_PREC)


def _round(res, cb):
    n = jnp.sum(cb * cb, axis=1)[None, :]
    s = 2.0 * _dot(res, cb, ((1,), (1,))) - n
    m = jnp.max(s, axis=1, keepdims=True)
    e = jnp.exp(s - m)
    d = jnp.sum(e, axis=1, keepdims=True)
    q = _dot(e, cb, ((1,), (0,))) * (1.0 / d)
    return q, m + jnp.log(d)


def _body(x_ref, cb_ref, c2_ref, pint_ref, recon_ref, loss_ref):
    x = x_ref[...]
    cb0 = cb_ref[0]
    cb1 = cb_ref[1]
    c2 = c2_ref[...]

    q0, l0 = _round(x, cb0)
    r1 = x - q0
    q1, l1 = _round(r1, cb1)
    recon = q0 + q1
    recon_ref[...] = recon
    dr = recon - x
    loss_ref[...] = (jnp.sum(r1 * r1) + jnp.sum(dr * dr)).reshape(1, 1, 1)

    a = jnp.concatenate([x, r1], axis=1)
    n_int = 0.25 * jnp.sum(c2 * c2, axis=0)[None, :]
    s = _dot(a, c2, ((1,), (0,)))
    lane = jax.lax.broadcasted_iota(jnp.int32, (1, 2 * _K), 1)
    even = ((lane >> 7) & 1) == 0
    lsel = jnp.where(even, l0, l1)
    p = jnp.exp(s - (n_int + lsel))
    pint_ref[...] = p.reshape(_TT, 128, 128)


@jax.jit
def kernel(x, codebooks):
    x2 = x.reshape(_BT, _D)
    cb = codebooks
    zg = jnp.zeros((_NG, 128, _D), dtype=cb.dtype)
    cb0g = cb[0].reshape(_NG, 128, _D)
    cb1g = cb[1].reshape(_NG, 128, _D)
    top = jnp.stack([cb0g, zg], axis=1).reshape(2 * _K, _D)
    bot = jnp.stack([zg, cb1g], axis=1).reshape(2 * _K, _D)
    c2 = 2.0 * jnp.concatenate([top, bot], axis=1).T

    pint, recon2, losses = pl.pallas_call(
        _body,
        grid=(_G,),
        in_specs=[
            pl.BlockSpec((_TT, _D), lambda i: (i, 0)),
            pl.BlockSpec((2, _K, _D), lambda i: (0, 0, 0)),
            pl.BlockSpec((2 * _D, 2 * _K), lambda i: (0, 0)),
        ],
        out_specs=[
            pl.BlockSpec((_TT, 128, 128), lambda i: (i, 0, 0)),
            pl.BlockSpec((_TT, _D), lambda i: (i, 0)),
            pl.BlockSpec((1, 1, 1), lambda i: (i, 0, 0)),
        ],
        out_shape=[
            jax.ShapeDtypeStruct((_BT, 128, 128), jnp.float32),
            jax.ShapeDtypeStruct((_BT, _D), jnp.float32),
            jax.ShapeDtypeStruct((_G, 1, 1), jnp.float32),
        ],
        compiler_params=pltpu.CompilerParams(
            dimension_semantics=("parallel",),
        ),
    )(x2, cb, c2)
    index_probs = (pint.reshape(_B, _T, _NG, 2, 128)
                   .transpose(0, 1, 2, 4, 3)
                   .reshape(_B, _T, _K, 2))
    recon = recon2.reshape(_B, _T, _D)
    loss = jnp.sum(losses) * (1.25 / (_BT * _D))
    return recon, index_probs, loss


# speedup vs baseline: 3.6656x; 1.0491x over previous
import functools

import jax
import jax.numpy as jnp
from jax.experimental import pallas as pl
from jax.experimental.pallas import tpu as pltpu

_B, _T, _D = 16, 1024, 32
_K = 8192
_TT = 128
_BT = _B * _T
_G = _BT // _TT
_NG = _K // 128

_PREC = jax.lax.Precision.DEFAULT


def _dot(a, b, dims):
    return jax.lax.dot_general(
        a, b, (dims, ((), ())),
        preferred_element_type=jnp.float32, precision=_PREC)


def _round(res2, cb, n):
    s = _dot(res2, cb, ((1,), (1,))) - n
    m = jnp.max(s, axis=1, keepdims=True)
    e = jnp.exp(s - m)
    d = jnp.sum(e, axis=1, keepdims=True)
    q = _dot(e, cb, ((1,), (0,))) * (1.0 / d)
    return q, m + jnp.log(d)


def _body(x_ref, cb_ref, c2_ref, pint_ref, recon_ref, loss_ref,
          n0_ref, n1_ref, nint_ref):
    x = x_ref[...]
    cb0 = cb_ref[0]
    cb1 = cb_ref[1]
    c2 = c2_ref[...]

    @pl.when(pl.program_id(0) == 0)
    def _():
        n0_ref[...] = jnp.sum(cb0 * cb0, axis=1)[None, :]
        n1_ref[...] = jnp.sum(cb1 * cb1, axis=1)[None, :]
        nint_ref[...] = 0.25 * jnp.sum(c2 * c2, axis=0)[None, :]

    q0, l0 = _round(x + x, cb0, n0_ref[...])
    r1 = x - q0
    q1, l1 = _round(r1 + r1, cb1, n1_ref[...])
    recon = q0 + q1
    recon_ref[...] = recon
    dr = recon - x
    loss_ref[...] = (jnp.sum(r1 * r1) + jnp.sum(dr * dr)).reshape(1, 1, 1)

    a = jnp.concatenate([x, r1], axis=1)
    s = _dot(a, c2, ((1,), (0,)))
    lane = jax.lax.broadcasted_iota(jnp.int32, (1, 2 * _K), 1)
    even = ((lane >> 7) & 1) == 0
    lsel = jnp.where(even, l0, l1)
    p = jnp.exp(s - (nint_ref[...] + lsel))
    pint_ref[...] = p.reshape(_TT, 128, 128)


@jax.jit
def kernel(x, codebooks):
    x2 = x.reshape(_BT, _D)
    cb = codebooks
    zg = jnp.zeros((_NG, 128, _D), dtype=cb.dtype)
    cb0g = cb[0].reshape(_NG, 128, _D)
    cb1g = cb[1].reshape(_NG, 128, _D)
    top = jnp.stack([cb0g, zg], axis=1).reshape(2 * _K, _D)
    bot = jnp.stack([zg, cb1g], axis=1).reshape(2 * _K, _D)
    c2 = 2.0 * jnp.concatenate([top, bot], axis=1).T

    pint, recon2, losses = pl.pallas_call(
        _body,
        grid=(_G,),
        in_specs=[
            pl.BlockSpec((_TT, _D), lambda i: (i, 0)),
            pl.BlockSpec((2, _K, _D), lambda i: (0, 0, 0)),
            pl.BlockSpec((2 * _D, 2 * _K), lambda i: (0, 0)),
        ],
        out_specs=[
            pl.BlockSpec((_TT, 128, 128), lambda i: (i, 0, 0)),
            pl.BlockSpec((_TT, _D), lambda i: (i, 0)),
            pl.BlockSpec((1, 1, 1), lambda i: (i, 0, 0)),
        ],
        out_shape=[
            jax.ShapeDtypeStruct((_BT, 128, 128), jnp.float32),
            jax.ShapeDtypeStruct((_BT, _D), jnp.float32),
            jax.ShapeDtypeStruct((_G, 1, 1), jnp.float32),
        ],
        scratch_shapes=[
            pltpu.VMEM((1, _K), jnp.float32),
            pltpu.VMEM((1, _K), jnp.float32),
            pltpu.VMEM((1, 2 * _K), jnp.float32),
        ],
        compiler_params=pltpu.CompilerParams(
            dimension_semantics=("arbitrary",),
        ),
    )(x2, cb, c2)
    index_probs = (pint.reshape(_B, _T, _NG, 2, 128)
                   .transpose(0, 1, 2, 4, 3)
                   .reshape(_B, _T, _K, 2))
    recon = recon2.reshape(_B, _T, _D)
    loss = jnp.sum(losses) * (1.25 / (_BT * _D))
    return recon, index_probs, loss


# speedup vs baseline: 4.3662x; 1.1911x over previous
import functools

import jax
import jax.numpy as jnp
from jax.experimental import pallas as pl
from jax.experimental.pallas import tpu as pltpu

_B, _T, _D = 16, 1024, 32
_K = 8192
_TT = 128
_BT = _B * _T
_G = _BT // _TT
_NG = _K // 128

_PREC = jax.lax.Precision.DEFAULT


def _dot(a, b, dims):
    return jax.lax.dot_general(
        a, b, (dims, ((), ())),
        preferred_element_type=jnp.float32, precision=_PREC)


def _round(res, cbn):
    res2m = jnp.concatenate(
        [res + res, jnp.full((_TT, 1), -1.0, jnp.float32)], axis=1)
    s = _dot(res2m, cbn, ((1,), (1,)))
    m = jnp.max(s, axis=1, keepdims=True)
    e = jnp.exp(s - m)
    d = jnp.sum(e, axis=1, keepdims=True)
    q = _dot(e, cbn[:, :_D], ((1,), (0,))) * (1.0 / d)
    return q, m + jnp.log(d)


def _body(x_ref, cbn_ref, c2_ref, pint_ref, recon_ref, loss_ref):
    x = x_ref[...]
    c2 = c2_ref[...]

    q0, l0 = _round(x, cbn_ref[0])
    r1 = x - q0
    q1, l1 = _round(r1, cbn_ref[1])
    recon = q0 + q1
    recon_ref[...] = recon
    dr = recon - x
    loss_ref[...] = (jnp.sum(r1 * r1) + jnp.sum(dr * dr)).reshape(1, 1, 1)

    a = jnp.concatenate(
        [x, r1, jnp.ones((_TT, 1), jnp.float32), l0, l1], axis=1)
    p = jnp.exp(_dot(a, c2, ((1,), (0,))))
    pint_ref[...] = p.reshape(_TT, 128, 128)


@jax.jit
def kernel(x, codebooks):
    x2 = x.reshape(_BT, _D)
    cb = codebooks
    nrm = jnp.sum(cb * cb, axis=-1, keepdims=True)
    cbn = jnp.concatenate([cb, nrm], axis=-1)

    zg = jnp.zeros((_NG, 128, _D), dtype=cb.dtype)
    cb0g = cb[0].reshape(_NG, 128, _D)
    cb1g = cb[1].reshape(_NG, 128, _D)
    top = 2.0 * jnp.stack([cb0g, zg], axis=1).reshape(2 * _K, _D)
    bot = 2.0 * jnp.stack([zg, cb1g], axis=1).reshape(2 * _K, _D)
    n_int = jnp.stack(
        [nrm[0].reshape(_NG, 128), nrm[1].reshape(_NG, 128)],
        axis=1).reshape(2 * _K, 1)
    rbit = (jnp.arange(2 * _K, dtype=jnp.int32)[:, None] >> 7) & 1
    mask_e = jnp.where(rbit == 0, -1.0, 0.0).astype(jnp.float32)
    mask_o = jnp.where(rbit == 0, 0.0, -1.0).astype(jnp.float32)
    c2 = jnp.concatenate([top, bot, -n_int, mask_e, mask_o], axis=1).T

    pint, recon2, losses = pl.pallas_call(
        _body,
        grid=(_G,),
        in_specs=[
            pl.BlockSpec((_TT, _D), lambda i: (i, 0)),
            pl.BlockSpec((2, _K, _D + 1), lambda i: (0, 0, 0)),
            pl.BlockSpec((2 * _D + 3, 2 * _K), lambda i: (0, 0)),
        ],
        out_specs=[
            pl.BlockSpec((_TT, 128, 128), lambda i: (i, 0, 0)),
            pl.BlockSpec((_TT, _D), lambda i: (i, 0)),
            pl.BlockSpec((1, 1, 1), lambda i: (i, 0, 0)),
        ],
        out_shape=[
            jax.ShapeDtypeStruct((_BT, 128, 128), jnp.float32),
            jax.ShapeDtypeStruct((_BT, _D), jnp.float32),
            jax.ShapeDtypeStruct((_G, 1, 1), jnp.float32),
        ],
        compiler_params=pltpu.CompilerParams(
            dimension_semantics=("arbitrary",),
        ),
    )(x2, cbn, c2)
    index_probs = (pint.reshape(_B, _T, _NG, 2, 128)
                   .transpose(0, 1, 2, 4, 3)
                   .reshape(_B, _T, _K, 2))
    recon = recon2.reshape(_B, _T, _D)
    loss = jnp.sum(losses) * (1.25 / (_BT * _D))
    return recon, index_probs, loss
